# baseline (speedup 1.0000x reference)
import functools
import math

import jax
import jax.numpy as jnp
from jax.experimental import pallas as pl
from jax.experimental.pallas import tpu as pltpu

_EPS = 1e-2
_NUM_ITERS = 10
_SCALE = 1.0 / (_EPS * math.log(2.0))
_SQRT_SCALE = math.sqrt(_SCALE)
_SHIFT = 44.0
_TB = 8
_UNROLL = 8


def _softranks_body(x_ref, o_ref, c_ref, ct_ref, acc_ref, wacc_ref, zc_ref,
                    br1_ref, br2_ref, *, n: int):
    nb = n // _TB
    nl = n // 128
    log2n = math.log2(n)
    y_step = _SQRT_SCALE / (n - 1)

    x = x_ref[...].reshape(1, n)
    xmin = jnp.min(x)
    xmax = jnp.max(x)
    zs_row = (x - xmin) * (_SQRT_SCALE / (xmax - xmin + 1e-12))
    iota_row = jax.lax.broadcasted_iota(jnp.int32, (1, n), 1).astype(jnp.float32)
    zc_ref[...] = zs_row.reshape(n, 1)

    def lanefold(e):
        parts = [e[:, k * 128:(k + 1) * 128] for k in range(nl)]
        while len(parts) > 1:
            parts = [parts[i] + parts[i + 1] for i in range(0, len(parts), 2)]
        return parts[0]

    def finish_pass(ref):
        s_col = jnp.sum(ref[...], axis=1, keepdims=True)
        return s_col.reshape(1, n)

    def tb_iota_col(b):
        i8 = jax.lax.broadcasted_iota(jnp.int32, (_TB, 1), 0)
        return (i8 + b * _TB).astype(jnp.float32)

    br1_ref[...] = jnp.broadcast_to(zs_row, (_TB, n))

    def build_ct_blk(b, _):
        ycol = tb_iota_col(b) * y_step
        tile = (ycol - br1_ref[...]) ** 2
        ct_ref[pl.ds(b * _TB, _TB), :] = tile
        acc_ref[pl.ds(b * _TB, _TB), :] = lanefold(jnp.exp2(_SHIFT - tile))
        return 0

    jax.lax.fori_loop(0, nb, build_ct_blk, 0, unroll=_UNROLL)
    gamma_row = (-log2n + _SHIFT) - jnp.log2(finish_pass(acc_ref))

    u2 = jnp.max(gamma_row) - _SHIFT
    br1_ref[...] = jnp.broadcast_to(iota_row * y_step, (_TB, n))
    br2_ref[...] = jnp.broadcast_to(gamma_row - u2, (_TB, n))

    def build_c_blk(b, _):
        zcol = zc_ref[pl.ds(b * _TB, _TB), :]
        tile = (zcol - br1_ref[...]) ** 2
        c_ref[pl.ds(b * _TB, _TB), :] = tile
        acc_ref[pl.ds(b * _TB, _TB), :] = lanefold(jnp.exp2(br2_ref[...] - tile))
        return 0

    jax.lax.fori_loop(0, nb, build_c_blk, 0, unroll=_UNROLL)
    phi_row = (-log2n - u2) - jnp.log2(finish_pass(acc_ref))

    def sumexp2(m_ref):
        def blk(b, _):
            tile = m_ref[pl.ds(b * _TB, _TB), :]
            acc_ref[pl.ds(b * _TB, _TB), :] = lanefold(
                jnp.exp2(br1_ref[...] - tile))
            return 0

        jax.lax.fori_loop(0, nb, blk, 0, unroll=_UNROLL)
        return finish_pass(acc_ref)

    def step(_, carry):
        phi_row, _ = carry
        u1 = jnp.max(phi_row) - _SHIFT
        br1_ref[...] = jnp.broadcast_to(phi_row - u1, (_TB, n))
        gamma_row = (-log2n - u1) - jnp.log2(sumexp2(ct_ref))
        u2 = jnp.max(gamma_row) - _SHIFT
        br1_ref[...] = jnp.broadcast_to(gamma_row - u2, (_TB, n))
        phi_row = (-log2n - u2) - jnp.log2(sumexp2(c_ref))
        return phi_row, gamma_row

    phi_row, gamma_row = jax.lax.fori_loop(
        0, _NUM_ITERS - 2, step, (phi_row, gamma_row)
    )

    u1 = jnp.max(phi_row) - _SHIFT
    br1_ref[...] = jnp.broadcast_to(phi_row - u1, (_TB, n))
    gamma_row = (-log2n - u1) - jnp.log2(sumexp2(ct_ref))
    u2 = jnp.max(gamma_row) - _SHIFT
    br1_ref[...] = jnp.broadcast_to(gamma_row - u2, (_TB, n))
    br2_ref[...] = jnp.broadcast_to(iota_row + 1.0, (_TB, n))

    def last_blk(b, _):
        tile = c_ref[pl.ds(b * _TB, _TB), :]
        e = jnp.exp2(br1_ref[...] - tile)
        acc_ref[pl.ds(b * _TB, _TB), :] = lanefold(e)
        wacc_ref[pl.ds(b * _TB, _TB), :] = lanefold(e * br2_ref[...])
        return 0

    jax.lax.fori_loop(0, nb, last_blk, 0, unroll=_UNROLL)
    s_col = jnp.sum(acc_ref[...], axis=1, keepdims=True)
    t_col = jnp.sum(wacc_ref[...], axis=1, keepdims=True)
    ranks_col = t_col / s_col - 1.0
    o_ref[...] = ranks_col.reshape(1, 1, n)


@jax.jit
def kernel(inputs):
    b, n = inputs.shape
    out = pl.pallas_call(
        functools.partial(_softranks_body, n=n),
        grid=(b,),
        in_specs=[pl.BlockSpec((1, 1, n), lambda i: (i, 0, 0))],
        out_specs=pl.BlockSpec((1, 1, n), lambda i: (i, 0, 0)),
        out_shape=jax.ShapeDtypeStruct((b, 1, n), jnp.float32),
        scratch_shapes=[
            pltpu.VMEM((n, n), jnp.float32),
            pltpu.VMEM((n, n), jnp.float32),
            pltpu.VMEM((n, 128), jnp.float32),
            pltpu.VMEM((n, 128), jnp.float32),
            pltpu.VMEM((n, 1), jnp.float32),
            pltpu.VMEM((_TB, n), jnp.float32),
            pltpu.VMEM((_TB, n), jnp.float32),
        ],
    )(inputs.reshape(b, 1, n))
    return out.reshape(b, n)


# speedup vs baseline: 1.4489x; 1.4489x over previous
import functools
import math

import jax
import jax.numpy as jnp
from jax.experimental import pallas as pl

_EPS = 1e-2
_NUM_ITERS = 10
_SCALE = 1.0 / (_EPS * math.log(2.0))
_SQRT_SCALE = math.sqrt(_SCALE)
_SHIFT = 44.0


def _softranks_body(x_ref, o_ref, *, n: int):
    log2n = math.log2(n)
    x = x_ref[...].reshape(1, n)
    xmin = jnp.min(x)
    xmax = jnp.max(x)
    zs_row = (x - xmin) * (_SQRT_SCALE / (xmax - xmin + 1e-12))
    iota_row = jax.lax.broadcasted_iota(jnp.int32, (1, n), 1).astype(jnp.float32)
    ys_row = iota_row * (_SQRT_SCALE / (n - 1))
    zs_col = zs_row.reshape(n, 1)
    c = (zs_col - ys_row) ** 2

    def g_update(phi_col):
        u1 = jnp.max(phi_col) - _SHIFT
        s1 = jnp.sum(jnp.exp2((phi_col - u1) - c), axis=0, keepdims=True)
        return (-log2n - u1) - jnp.log2(s1)

    def step(_, carry):
        phi_col, _ = carry
        gamma_row = g_update(phi_col)
        u2 = jnp.max(gamma_row) - _SHIFT
        s2 = jnp.sum(jnp.exp2((gamma_row - u2) - c), axis=1, keepdims=True)
        phi_col = (-log2n - u2) - jnp.log2(s2)
        return phi_col, gamma_row

    zcol = jnp.zeros((n, 1), jnp.float32)
    zrow = jnp.zeros((1, n), jnp.float32)
    phi_col, _ = jax.lax.fori_loop(0, _NUM_ITERS - 1, step, (zcol, zrow))

    gamma_row = g_update(phi_col)
    u2 = jnp.max(gamma_row) - _SHIFT
    e2 = jnp.exp2((gamma_row - u2) - c)
    s2 = jnp.sum(e2, axis=1, keepdims=True)
    t2 = jnp.sum(e2 * (iota_row + 1.0), axis=1, keepdims=True)
    ranks_col = t2 / s2 - 1.0
    o_ref[...] = ranks_col.reshape(1, 1, n)


@jax.jit
def kernel(inputs):
    b, n = inputs.shape
    out = pl.pallas_call(
        functools.partial(_softranks_body, n=n),
        grid=(b,),
        in_specs=[pl.BlockSpec((1, 1, n), lambda i: (i, 0, 0))],
        out_specs=pl.BlockSpec((1, 1, n), lambda i: (i, 0, 0)),
        out_shape=jax.ShapeDtypeStruct((b, 1, n), jnp.float32),
    )(inputs.reshape(b, 1, n))
    return out.reshape(b, n)


# speedup vs baseline: 1.8782x; 1.2963x over previous
import functools
import math

import jax
import jax.numpy as jnp
from jax.experimental import pallas as pl

_EPS = 1e-2
_NUM_ITERS = 10
_SCALE = 1.0 / (_EPS * math.log(2.0))
_SQRT_SCALE = math.sqrt(_SCALE)
_SHIFT = 44.0


def _softranks_body(x_ref, o_ref, *, n: int):
    log2n = math.log2(n)
    x = x_ref[...].reshape(1, n)
    xmin = jnp.min(x)
    xmax = jnp.max(x)
    zs_row = (x - xmin) * (_SQRT_SCALE / (xmax - xmin + 1e-12))
    iota_row = jax.lax.broadcasted_iota(jnp.int32, (1, n), 1).astype(jnp.float32)
    ys_row = iota_row * (_SQRT_SCALE / (n - 1))
    zs_col = zs_row.reshape(n, 1)
    e_mat = jnp.exp2(-((zs_col - ys_row) ** 2))

    def g_update(phi_col):
        u1 = jnp.max(phi_col) - _SHIFT
        w1 = jnp.exp2(phi_col - u1)
        s1 = jnp.sum(w1 * e_mat, axis=0, keepdims=True)
        return (-log2n - u1) - jnp.log2(s1)

    def step(_, carry):
        phi_col, _ = carry
        gamma_row = g_update(phi_col)
        u2 = jnp.max(gamma_row) - _SHIFT
        w2 = jnp.exp2(gamma_row - u2)
        s2 = jnp.sum(e_mat * w2, axis=1, keepdims=True)
        phi_col = (-log2n - u2) - jnp.log2(s2)
        return phi_col, gamma_row

    zcol = jnp.zeros((n, 1), jnp.float32)
    zrow = jnp.zeros((1, n), jnp.float32)
    phi_col, _ = jax.lax.fori_loop(0, _NUM_ITERS - 1, step, (zcol, zrow))

    gamma_row = g_update(phi_col)
    u2 = jnp.max(gamma_row) - _SHIFT
    w2 = jnp.exp2(gamma_row - u2)
    p2 = e_mat * w2
    s2 = jnp.sum(p2, axis=1, keepdims=True)
    t2 = jnp.sum(p2 * (iota_row + 1.0), axis=1, keepdims=True)
    ranks_col = t2 / s2 - 1.0
    o_ref[...] = ranks_col.reshape(1, 1, n)


@jax.jit
def kernel(inputs):
    b, n = inputs.shape
    out = pl.pallas_call(
        functools.partial(_softranks_body, n=n),
        grid=(b,),
        in_specs=[pl.BlockSpec((1, 1, n), lambda i: (i, 0, 0))],
        out_specs=pl.BlockSpec((1, 1, n), lambda i: (i, 0, 0)),
        out_shape=jax.ShapeDtypeStruct((b, 1, n), jnp.float32),
    )(inputs.reshape(b, 1, n))
    return out.reshape(b, n)


# speedup vs baseline: 2.0392x; 1.0857x over previous
import functools
import math

import jax
import jax.numpy as jnp
from jax.experimental import pallas as pl

_EPS = 1e-2
_NUM_ITERS = 10
_SCALE = 1.0 / (_EPS * math.log(2.0))
_SQRT_SCALE = math.sqrt(_SCALE)
_SHIFT = 44.0


_ROWS = 2


def _softranks_body(x_ref, o_ref, *, n: int):
    log2n = math.log2(n)
    iota_row = jax.lax.broadcasted_iota(jnp.int32, (1, n), 1).astype(jnp.float32)
    ys_row = iota_row * (_SQRT_SCALE / (n - 1))

    def make_e(r):
        x = x_ref[r].reshape(1, n)
        xmin = jnp.min(x)
        xmax = jnp.max(x)
        zs_row = (x - xmin) * (_SQRT_SCALE / (xmax - xmin + 1e-12))
        zs_col = zs_row.reshape(n, 1)
        return jnp.exp2(-((zs_col - ys_row) ** 2))

    e_mats = [make_e(r) for r in range(_ROWS)]

    def g_update(e_mat, phi_col):
        u1 = jnp.max(phi_col) - _SHIFT
        w1 = jnp.exp2(phi_col - u1)
        s1 = jnp.sum(w1 * e_mat, axis=0, keepdims=True)
        return (-log2n - u1) - jnp.log2(s1)

    def f_update(e_mat, gamma_row):
        u2 = jnp.max(gamma_row) - _SHIFT
        w2 = jnp.exp2(gamma_row - u2)
        s2 = jnp.sum(e_mat * w2, axis=1, keepdims=True)
        return (-log2n - u2) - jnp.log2(s2)

    def step(_, carry):
        out = []
        for r in range(_ROWS):
            phi_col, _ = carry[r]
            gamma_row = g_update(e_mats[r], phi_col)
            phi_col = f_update(e_mats[r], gamma_row)
            out.append((phi_col, gamma_row))
        return tuple(out)

    zcol = jnp.zeros((n, 1), jnp.float32)
    zrow = jnp.zeros((1, n), jnp.float32)
    carry = jax.lax.fori_loop(
        0, _NUM_ITERS - 1, step, tuple((zcol, zrow) for _ in range(_ROWS))
    )

    for r in range(_ROWS):
        phi_col, _ = carry[r]
        gamma_row = g_update(e_mats[r], phi_col)
        u2 = jnp.max(gamma_row) - _SHIFT
        w2 = jnp.exp2(gamma_row - u2)
        p2 = e_mats[r] * w2
        s2 = jnp.sum(p2, axis=1, keepdims=True)
        t2 = jnp.sum(p2 * (iota_row + 1.0), axis=1, keepdims=True)
        s2_row = s2.reshape(1, n)
        t2_row = t2.reshape(1, n)
        o_ref[r] = (t2_row / s2_row - 1.0).reshape(1, n)


@jax.jit
def kernel(inputs):
    b, n = inputs.shape
    out = pl.pallas_call(
        functools.partial(_softranks_body, n=n),
        grid=(b // _ROWS,),
        in_specs=[pl.BlockSpec((_ROWS, 1, n), lambda i: (i, 0, 0))],
        out_specs=pl.BlockSpec((_ROWS, 1, n), lambda i: (i, 0, 0)),
        out_shape=jax.ShapeDtypeStruct((b, 1, n), jnp.float32),
    )(inputs.reshape(b, 1, n))
    return out.reshape(b, n)


# speedup vs baseline: 2.1213x; 1.0403x over previous
import functools
import math

import jax
import jax.numpy as jnp
from jax.experimental import pallas as pl

_EPS = 1e-2
_NUM_ITERS = 10
_SCALE = 1.0 / (_EPS * math.log(2.0))
_SQRT_SCALE = math.sqrt(_SCALE)
_SHIFT = 44.0


_ROWS = 4


def _softranks_body(x_ref, o_ref, *, n: int):
    log2n = math.log2(n)
    iota_row = jax.lax.broadcasted_iota(jnp.int32, (1, n), 1).astype(jnp.float32)
    ys_row = iota_row * (_SQRT_SCALE / (n - 1))

    def make_e(r):
        x = x_ref[r].reshape(1, n)
        xmin = jnp.min(x)
        xmax = jnp.max(x)
        zs_row = (x - xmin) * (_SQRT_SCALE / (xmax - xmin + 1e-12))
        zs_col = zs_row.reshape(n, 1)
        return jnp.exp2(-((zs_col - ys_row) ** 2))

    e_mats = [make_e(r) for r in range(_ROWS)]

    def g_update(e_mat, phi_col):
        u1 = jnp.max(phi_col) - _SHIFT
        w1 = jnp.exp2(phi_col - u1)
        s1 = jnp.sum(w1 * e_mat, axis=0, keepdims=True)
        return (-log2n - u1) - jnp.log2(s1)

    def f_update(e_mat, gamma_row):
        u2 = jnp.max(gamma_row) - _SHIFT
        w2 = jnp.exp2(gamma_row - u2)
        s2 = jnp.sum(e_mat * w2, axis=1, keepdims=True)
        return (-log2n - u2) - jnp.log2(s2)

    def step(_, carry):
        out = []
        for r in range(_ROWS):
            phi_col, _ = carry[r]
            gamma_row = g_update(e_mats[r], phi_col)
            phi_col = f_update(e_mats[r], gamma_row)
            out.append((phi_col, gamma_row))
        return tuple(out)

    zcol = jnp.zeros((n, 1), jnp.float32)
    zrow = jnp.zeros((1, n), jnp.float32)
    carry = jax.lax.fori_loop(
        0, _NUM_ITERS - 1, step, tuple((zcol, zrow) for _ in range(_ROWS))
    )

    for r in range(_ROWS):
        phi_col, _ = carry[r]
        gamma_row = g_update(e_mats[r], phi_col)
        u2 = jnp.max(gamma_row) - _SHIFT
        w2 = jnp.exp2(gamma_row - u2)
        p2 = e_mats[r] * w2
        s2 = jnp.sum(p2, axis=1, keepdims=True)
        t2 = jnp.sum(p2 * (iota_row + 1.0), axis=1, keepdims=True)
        s2_row = s2.reshape(1, n)
        t2_row = t2.reshape(1, n)
        o_ref[r] = (t2_row / s2_row - 1.0).reshape(1, n)


@jax.jit
def kernel(inputs):
    b, n = inputs.shape
    out = pl.pallas_call(
        functools.partial(_softranks_body, n=n),
        grid=(b // _ROWS,),
        in_specs=[pl.BlockSpec((_ROWS, 1, n), lambda i: (i, 0, 0))],
        out_specs=pl.BlockSpec((_ROWS, 1, n), lambda i: (i, 0, 0)),
        out_shape=jax.ShapeDtypeStruct((b, 1, n), jnp.float32),
    )(inputs.reshape(b, 1, n))
    return out.reshape(b, n)


# speedup vs baseline: 2.2402x; 1.0560x over previous
import functools
import math

import jax
import jax.numpy as jnp
from jax.experimental import pallas as pl

_EPS = 1e-2
_NUM_ITERS = 10
_SCALE = 1.0 / (_EPS * math.log(2.0))
_SQRT_SCALE = math.sqrt(_SCALE)
_SHIFT = 44.0


_ROWS = 8


def _softranks_body(x_ref, o_ref, *, n: int):
    log2n = math.log2(n)
    iota_row = jax.lax.broadcasted_iota(jnp.int32, (1, n), 1).astype(jnp.float32)
    ys_row = iota_row * (_SQRT_SCALE / (n - 1))

    def make_e(r):
        x = x_ref[r].reshape(1, n)
        xmin = jnp.min(x)
        xmax = jnp.max(x)
        zs_row = (x - xmin) * (_SQRT_SCALE / (xmax - xmin + 1e-12))
        zs_col = zs_row.reshape(n, 1)
        return jnp.exp2(-((zs_col - ys_row) ** 2))

    e_mats = [make_e(r) for r in range(_ROWS)]

    def g_update(e_mat, phi_col):
        u1 = jnp.max(phi_col) - _SHIFT
        w1 = jnp.exp2(phi_col - u1)
        s1 = jnp.sum(w1 * e_mat, axis=0, keepdims=True)
        return (-log2n - u1) - jnp.log2(s1)

    def f_update(e_mat, gamma_row):
        u2 = jnp.max(gamma_row) - _SHIFT
        w2 = jnp.exp2(gamma_row - u2)
        s2 = jnp.sum(e_mat * w2, axis=1, keepdims=True)
        return (-log2n - u2) - jnp.log2(s2)

    def step(_, carry):
        out = []
        for r in range(_ROWS):
            phi_col, _ = carry[r]
            gamma_row = g_update(e_mats[r], phi_col)
            phi_col = f_update(e_mats[r], gamma_row)
            out.append((phi_col, gamma_row))
        return tuple(out)

    zcol = jnp.zeros((n, 1), jnp.float32)
    zrow = jnp.zeros((1, n), jnp.float32)
    carry = jax.lax.fori_loop(
        0, _NUM_ITERS - 1, step, tuple((zcol, zrow) for _ in range(_ROWS))
    )

    for r in range(_ROWS):
        phi_col, _ = carry[r]
        gamma_row = g_update(e_mats[r], phi_col)
        u2 = jnp.max(gamma_row) - _SHIFT
        w2 = jnp.exp2(gamma_row - u2)
        p2 = e_mats[r] * w2
        s2 = jnp.sum(p2, axis=1, keepdims=True)
        t2 = jnp.sum(p2 * (iota_row + 1.0), axis=1, keepdims=True)
        s2_row = s2.reshape(1, n)
        t2_row = t2.reshape(1, n)
        o_ref[r] = (t2_row / s2_row - 1.0).reshape(1, n)


@jax.jit
def kernel(inputs):
    b, n = inputs.shape
    out = pl.pallas_call(
        functools.partial(_softranks_body, n=n),
        grid=(b // _ROWS,),
        in_specs=[pl.BlockSpec((_ROWS, 1, n), lambda i: (i, 0, 0))],
        out_specs=pl.BlockSpec((_ROWS, 1, n), lambda i: (i, 0, 0)),
        out_shape=jax.ShapeDtypeStruct((b, 1, n), jnp.float32),
    )(inputs.reshape(b, 1, n))
    return out.reshape(b, n)
